# Initial kernel scaffold; baseline (speedup 1.0000x reference)
import jax
import jax.numpy as jnp
from jax.experimental import pallas as pl


def kernel(x, W1f, W1d, W2f, W2d, W3f, W3d, W4f, W4d, W5):
    raise NotImplementedError("write your pallas kernel here")


# speedup vs baseline: 3.3325x; 3.3325x over previous
import functools

import jax
import jax.numpy as jnp
from jax.experimental import pallas as pl

EPS = 1e-6
_B, _N, _K = 4, 2048, 20
_BNQ = 256
_BN = 512
_F32 = jnp.float32


def _knn_feat_kernel(x_ref, xT_ref, xq_ref, W1f_ref, feat_ref, s1_ref):
    b = pl.program_id(0)
    i = pl.program_id(1)

    @pl.when(jnp.logical_and(b == 0, i == 0))
    def _init():
        s1_ref[...] = jnp.zeros(s1_ref.shape, _F32)

    xall = x_ref[0]
    xTb = xT_ref[0]
    ctr = xq_ref[0]

    inner = jnp.dot(xall, ctr, preferred_element_type=_F32)
    sq_all = jnp.sum(xall * xall, axis=1, keepdims=True)
    sq_q = jnp.sum(ctr * ctr, axis=0, keepdims=True)
    pdT = (2.0 * inner - sq_q) - sq_all

    iota = jax.lax.broadcasted_iota(jnp.int32, (_N, _BNQ), 0)
    c = [ctr[0:1], ctr[1:2], ctr[2:3]]

    acc_s = jnp.zeros((64, 1), _F32)
    acc_q = jnp.zeros((64, 1), _F32)
    for j in range(_K):
        m = jnp.max(pdT, axis=0, keepdims=True)
        cand = jnp.where(pdT >= m, iota, _N)
        idx = jnp.min(cand, axis=0, keepdims=True)
        eq = iota == idx
        pdT = jnp.where(eq, -1e30, pdT)
        onehot = eq.astype(_F32)
        nbrT = jnp.dot(xTb, onehot, preferred_element_type=_F32,
                       precision=jax.lax.Precision.HIGHEST)
        n = [nbrT[0:1], nbrT[1:2], nbrT[2:3]]
        r = [n[1] * c[2] - n[2] * c[1],
             n[2] * c[0] - n[0] * c[2],
             n[0] * c[1] - n[1] * c[0]]
        f = jnp.concatenate(
            [n[0] - c[0], c[0], r[0],
             n[1] - c[1], c[1], r[1],
             n[2] - c[2], c[2], r[2]], axis=0)
        feat_ref[0, j] = f
        ps = [jnp.dot(W1f_ref[...], f[3 * d:3 * d + 3],
                      preferred_element_type=_F32) for d in range(3)]
        nsq = ps[0] * ps[0] + ps[1] * ps[1] + ps[2] * ps[2]
        nrm = jnp.sqrt(nsq) + EPS
        acc_s += jnp.sum(nrm, axis=1, keepdims=True)
        acc_q += jnp.sum(nrm * nrm, axis=1, keepdims=True)
    s1_ref[...] += jnp.concatenate([acc_s, acc_q], axis=1)


def _bn_coefs(s_ref, cnt):
    mean = s_ref[:, 0:1] / cnt
    var = s_ref[:, 1:2] / cnt - mean * mean
    return mean, jnp.sqrt(var + 1e-5)


def _lbr(ps, ds, mean, denom):
    nsq = ps[0] * ps[0] + ps[1] * ps[1] + ps[2] * ps[2]
    nrm = jnp.sqrt(nsq) + EPS
    bn = (nrm - mean) / denom
    ps = [p / nrm * bn for p in ps]
    dot = ps[0] * ds[0] + ps[1] * ds[1] + ps[2] * ds[2]
    dsq = ds[0] * ds[0] + ds[1] * ds[1] + ds[2] * ds[2]
    coef = jnp.where(dot >= 0, 0.0, dot / (dsq + EPS))
    return [ps[d] - coef * ds[d] for d in range(3)]


def _norm_stats(ps):
    nsq = ps[0] * ps[0] + ps[1] * ps[1] + ps[2] * ps[2]
    nrm = jnp.sqrt(nsq) + EPS
    return jnp.concatenate(
        [jnp.sum(nrm, axis=1, keepdims=True),
         jnp.sum(nrm * nrm, axis=1, keepdims=True)], axis=1)


def _layer1_kernel(feat_ref, s1_ref, W1f_ref, W1d_ref, W2f_ref, h1_ref, s2_ref):
    b = pl.program_id(0)
    i = pl.program_id(1)

    @pl.when(jnp.logical_and(b == 0, i == 0))
    def _init():
        s2_ref[...] = jnp.zeros(s2_ref.shape, _F32)

    mean, denom = _bn_coefs(s1_ref, float(_B * _N * _K))
    bn = feat_ref.shape[3]
    acc = [jnp.zeros((64, bn), _F32) for _ in range(3)]
    for j in range(_K):
        f = feat_ref[0, j]
        ps = [jnp.dot(W1f_ref[...], f[3 * d:3 * d + 3],
                      preferred_element_type=_F32) for d in range(3)]
        ds = [jnp.dot(W1d_ref[...], f[3 * d:3 * d + 3],
                      preferred_element_type=_F32) for d in range(3)]
        out = _lbr(ps, ds, mean, denom)
        for d in range(3):
            acc[d] += out[d]
    hs = [a / float(_K) for a in acc]
    for d in range(3):
        h1_ref[0, d] = hs[d]
    p2 = [jnp.dot(W2f_ref[...], hs[d], preferred_element_type=_F32)
          for d in range(3)]
    s2_ref[...] += _norm_stats(p2)


def _mid_kernel(h_ref, s_ref, Wf_ref, Wd_ref, Wn_ref, ho_ref, sn_ref, *, cnt):
    b = pl.program_id(0)
    i = pl.program_id(1)

    @pl.when(jnp.logical_and(b == 0, i == 0))
    def _init():
        sn_ref[...] = jnp.zeros(sn_ref.shape, _F32)

    mean, denom = _bn_coefs(s_ref, cnt)
    hs = [h_ref[0, d] for d in range(3)]
    ps = [jnp.dot(Wf_ref[...], hs[d], preferred_element_type=_F32)
          for d in range(3)]
    ds = [jnp.dot(Wd_ref[...], hs[d], preferred_element_type=_F32)
          for d in range(3)]
    out = _lbr(ps, ds, mean, denom)
    for d in range(3):
        ho_ref[0, d] = out[d]
    pn = [jnp.dot(Wn_ref[...], out[d], preferred_element_type=_F32)
          for d in range(3)]
    sn_ref[...] += _norm_stats(pn)


def _final_kernel(h_ref, s5_ref, W5_ref, out_ref, *, cnt):
    i = pl.program_id(1)

    @pl.when(i == 0)
    def _init():
        out_ref[...] = jnp.zeros(out_ref.shape, _F32)

    mean, denom = _bn_coefs(s5_ref, cnt)
    hs = [h_ref[0, d] for d in range(3)]
    ps = [jnp.dot(W5_ref[...], hs[d], preferred_element_type=_F32)
          for d in range(3)]
    nsq = ps[0] * ps[0] + ps[1] * ps[1] + ps[2] * ps[2]
    nrm = jnp.sqrt(nsq) + EPS
    bn = (nrm - mean) / denom
    for d in range(3):
        col = jnp.sum(ps[d] / nrm * bn, axis=1, keepdims=True) / float(_N)
        out_ref[0, :, d:d + 1] += col


def kernel(x, W1f, W1d, W2f, W2d, W3f, W3d, W4f, W4d, W5):
    xT = jnp.transpose(x, (0, 2, 1))

    full = lambda shape: pl.BlockSpec(shape, lambda b, i: (0,) * len(shape))
    perb = lambda shape: pl.BlockSpec(
        shape, lambda b, i: (b,) + (0,) * (len(shape) - 1))

    feat, s1 = pl.pallas_call(
        _knn_feat_kernel,
        grid=(_B, _N // _BNQ),
        in_specs=[
            perb((1, _N, 3)),
            perb((1, 3, _N)),
            pl.BlockSpec((1, 3, _BNQ), lambda b, i: (b, 0, i)),
            full((64, 3)),
        ],
        out_specs=[
            pl.BlockSpec((1, _K, 9, _BNQ), lambda b, i: (b, 0, 0, i)),
            full((64, 2)),
        ],
        out_shape=[
            jax.ShapeDtypeStruct((_B, _K, 9, _N), _F32),
            jax.ShapeDtypeStruct((64, 2), _F32),
        ],
    )(x, xT, xT, W1f)

    grid = (_B, _N // _BN)
    hblk = lambda c: pl.BlockSpec((1, 3, c, _BN), lambda b, i: (b, 0, 0, i))

    h1, s2 = pl.pallas_call(
        _layer1_kernel,
        grid=grid,
        in_specs=[
            pl.BlockSpec((1, _K, 9, _BN), lambda b, i: (b, 0, 0, i)),
            full((64, 2)), full((64, 3)), full((64, 3)), full((64, 64)),
        ],
        out_specs=[hblk(64), full((64, 2))],
        out_shape=[
            jax.ShapeDtypeStruct((_B, 3, 64, _N), _F32),
            jax.ShapeDtypeStruct((64, 2), _F32),
        ],
    )(feat, s1, W1f, W1d, W2f)

    def mid(h, s, Wf, Wd, Wn, cin, cout, cnext):
        return pl.pallas_call(
            functools.partial(_mid_kernel, cnt=float(_B * _N)),
            grid=grid,
            in_specs=[
                hblk(cin), full((cout, 2)),
                full((cout, cin)), full((cout, cin)), full((cnext, cout)),
            ],
            out_specs=[hblk(cout), full((cnext, 2))],
            out_shape=[
                jax.ShapeDtypeStruct((_B, 3, cout, _N), _F32),
                jax.ShapeDtypeStruct((cnext, 2), _F32),
            ],
        )(h, s, Wf, Wd, Wn)

    h2, s3 = mid(h1, s2, W2f, W2d, W3f, 64, 64, 64)
    h3, s4 = mid(h2, s3, W3f, W3d, W4f, 64, 64, 128)
    h4, s5 = mid(h3, s4, W4f, W4d, W5, 64, 128, 1024)

    out = pl.pallas_call(
        functools.partial(_final_kernel, cnt=float(_B * _N)),
        grid=grid,
        in_specs=[hblk(128), full((1024, 2)), full((1024, 128))],
        out_specs=pl.BlockSpec((1, 1024, 3), lambda b, i: (b, 0, 0)),
        out_shape=jax.ShapeDtypeStruct((_B, 1024, 3), _F32),
    )(h4, s5, W5)
    return out


# speedup vs baseline: 4.0307x; 1.2095x over previous
import functools

import jax
import jax.numpy as jnp
from jax.experimental import pallas as pl

EPS = 1e-6
_B, _N, _K = 4, 2048, 20
_BNQ = 256
_BN = 512
_F32 = jnp.float32


def _knn_feat_kernel(x_ref, xT_ref, xq_ref, W1f_ref, feat_ref, s1_ref):
    b = pl.program_id(0)
    i = pl.program_id(1)

    @pl.when(jnp.logical_and(b == 0, i == 0))
    def _init():
        s1_ref[...] = jnp.zeros(s1_ref.shape, _F32)

    xall = x_ref[0]
    xTb = xT_ref[0]
    ctr = xq_ref[0]

    inner = jnp.dot(xall, ctr, preferred_element_type=_F32)
    sq_all = jnp.sum(xall * xall, axis=1, keepdims=True)
    sq_q = jnp.sum(ctr * ctr, axis=0, keepdims=True)
    pdT = (2.0 * inner - sq_q) - sq_all

    iota = jax.lax.broadcasted_iota(jnp.int32, (_N, _BNQ), 0)
    c = [ctr[0:1], ctr[1:2], ctr[2:3]]

    acc_s = jnp.zeros((64, 1), _F32)
    acc_q = jnp.zeros((64, 1), _F32)
    for j in range(_K):
        idx = jnp.argmax(pdT, axis=0)[None, :]
        eq = iota == idx
        pdT = jnp.where(eq, -1e30, pdT)
        onehot = eq.astype(_F32)
        nbrT = jnp.dot(xTb, onehot, preferred_element_type=_F32,
                       precision=jax.lax.Precision.HIGHEST)
        n = [nbrT[0:1], nbrT[1:2], nbrT[2:3]]
        r = [n[1] * c[2] - n[2] * c[1],
             n[2] * c[0] - n[0] * c[2],
             n[0] * c[1] - n[1] * c[0]]
        f = jnp.concatenate(
            [n[0] - c[0], c[0], r[0],
             n[1] - c[1], c[1], r[1],
             n[2] - c[2], c[2], r[2]], axis=0)
        feat_ref[0, j] = f
        ps = [jnp.dot(W1f_ref[...], f[3 * d:3 * d + 3],
                      preferred_element_type=_F32) for d in range(3)]
        nsq = ps[0] * ps[0] + ps[1] * ps[1] + ps[2] * ps[2]
        nrm = jnp.sqrt(nsq) + EPS
        acc_s += jnp.sum(nrm, axis=1, keepdims=True)
        acc_q += jnp.sum(nrm * nrm, axis=1, keepdims=True)
    s1_ref[...] += jnp.concatenate([acc_s, acc_q], axis=1)


def _bn_coefs(s_ref, cnt):
    mean = s_ref[:, 0:1] / cnt
    var = s_ref[:, 1:2] / cnt - mean * mean
    return mean, jnp.sqrt(var + 1e-5)


def _lbr(ps, ds, mean, denom):
    nsq = ps[0] * ps[0] + ps[1] * ps[1] + ps[2] * ps[2]
    nrm = jnp.sqrt(nsq) + EPS
    bn = (nrm - mean) / denom
    ps = [p / nrm * bn for p in ps]
    dot = ps[0] * ds[0] + ps[1] * ds[1] + ps[2] * ds[2]
    dsq = ds[0] * ds[0] + ds[1] * ds[1] + ds[2] * ds[2]
    coef = jnp.where(dot >= 0, 0.0, dot / (dsq + EPS))
    return [ps[d] - coef * ds[d] for d in range(3)]


def _norm_stats(ps):
    nsq = ps[0] * ps[0] + ps[1] * ps[1] + ps[2] * ps[2]
    nrm = jnp.sqrt(nsq) + EPS
    return jnp.concatenate(
        [jnp.sum(nrm, axis=1, keepdims=True),
         jnp.sum(nrm * nrm, axis=1, keepdims=True)], axis=1)


def _layer1_kernel(feat_ref, s1_ref, W1f_ref, W1d_ref, W2f_ref, h1_ref, s2_ref):
    b = pl.program_id(0)
    i = pl.program_id(1)

    @pl.when(jnp.logical_and(b == 0, i == 0))
    def _init():
        s2_ref[...] = jnp.zeros(s2_ref.shape, _F32)

    mean, denom = _bn_coefs(s1_ref, float(_B * _N * _K))
    bn = feat_ref.shape[3]
    acc = [jnp.zeros((64, bn), _F32) for _ in range(3)]
    for j in range(_K):
        f = feat_ref[0, j]
        ps = [jnp.dot(W1f_ref[...], f[3 * d:3 * d + 3],
                      preferred_element_type=_F32) for d in range(3)]
        ds = [jnp.dot(W1d_ref[...], f[3 * d:3 * d + 3],
                      preferred_element_type=_F32) for d in range(3)]
        out = _lbr(ps, ds, mean, denom)
        for d in range(3):
            acc[d] += out[d]
    hs = [a / float(_K) for a in acc]
    for d in range(3):
        h1_ref[0, d] = hs[d]
    p2 = [jnp.dot(W2f_ref[...], hs[d], preferred_element_type=_F32)
          for d in range(3)]
    s2_ref[...] += _norm_stats(p2)


def _mid_kernel(h_ref, s_ref, Wf_ref, Wd_ref, Wn_ref, ho_ref, sn_ref, *, cnt):
    b = pl.program_id(0)
    i = pl.program_id(1)

    @pl.when(jnp.logical_and(b == 0, i == 0))
    def _init():
        sn_ref[...] = jnp.zeros(sn_ref.shape, _F32)

    mean, denom = _bn_coefs(s_ref, cnt)
    hs = [h_ref[0, d] for d in range(3)]
    ps = [jnp.dot(Wf_ref[...], hs[d], preferred_element_type=_F32)
          for d in range(3)]
    ds = [jnp.dot(Wd_ref[...], hs[d], preferred_element_type=_F32)
          for d in range(3)]
    out = _lbr(ps, ds, mean, denom)
    for d in range(3):
        ho_ref[0, d] = out[d]
    pn = [jnp.dot(Wn_ref[...], out[d], preferred_element_type=_F32)
          for d in range(3)]
    sn_ref[...] += _norm_stats(pn)


def _final_kernel(h_ref, s5_ref, W5_ref, out_ref, *, cnt):
    i = pl.program_id(1)

    @pl.when(i == 0)
    def _init():
        out_ref[...] = jnp.zeros(out_ref.shape, _F32)

    mean, denom = _bn_coefs(s5_ref, cnt)
    hs = [h_ref[0, d] for d in range(3)]
    ps = [jnp.dot(W5_ref[...], hs[d], preferred_element_type=_F32)
          for d in range(3)]
    nsq = ps[0] * ps[0] + ps[1] * ps[1] + ps[2] * ps[2]
    nrm = jnp.sqrt(nsq) + EPS
    bn = (nrm - mean) / denom
    for d in range(3):
        col = jnp.sum(ps[d] / nrm * bn, axis=1, keepdims=True) / float(_N)
        out_ref[0, :, d:d + 1] += col


def kernel(x, W1f, W1d, W2f, W2d, W3f, W3d, W4f, W4d, W5):
    xT = jnp.transpose(x, (0, 2, 1))

    full = lambda shape: pl.BlockSpec(shape, lambda b, i: (0,) * len(shape))
    perb = lambda shape: pl.BlockSpec(
        shape, lambda b, i: (b,) + (0,) * (len(shape) - 1))

    feat, s1 = pl.pallas_call(
        _knn_feat_kernel,
        grid=(_B, _N // _BNQ),
        in_specs=[
            perb((1, _N, 3)),
            perb((1, 3, _N)),
            pl.BlockSpec((1, 3, _BNQ), lambda b, i: (b, 0, i)),
            full((64, 3)),
        ],
        out_specs=[
            pl.BlockSpec((1, _K, 9, _BNQ), lambda b, i: (b, 0, 0, i)),
            full((64, 2)),
        ],
        out_shape=[
            jax.ShapeDtypeStruct((_B, _K, 9, _N), _F32),
            jax.ShapeDtypeStruct((64, 2), _F32),
        ],
    )(x, xT, xT, W1f)

    grid = (_B, _N // _BN)
    hblk = lambda c: pl.BlockSpec((1, 3, c, _BN), lambda b, i: (b, 0, 0, i))

    h1, s2 = pl.pallas_call(
        _layer1_kernel,
        grid=grid,
        in_specs=[
            pl.BlockSpec((1, _K, 9, _BN), lambda b, i: (b, 0, 0, i)),
            full((64, 2)), full((64, 3)), full((64, 3)), full((64, 64)),
        ],
        out_specs=[hblk(64), full((64, 2))],
        out_shape=[
            jax.ShapeDtypeStruct((_B, 3, 64, _N), _F32),
            jax.ShapeDtypeStruct((64, 2), _F32),
        ],
    )(feat, s1, W1f, W1d, W2f)

    def mid(h, s, Wf, Wd, Wn, cin, cout, cnext):
        return pl.pallas_call(
            functools.partial(_mid_kernel, cnt=float(_B * _N)),
            grid=grid,
            in_specs=[
                hblk(cin), full((cout, 2)),
                full((cout, cin)), full((cout, cin)), full((cnext, cout)),
            ],
            out_specs=[hblk(cout), full((cnext, 2))],
            out_shape=[
                jax.ShapeDtypeStruct((_B, 3, cout, _N), _F32),
                jax.ShapeDtypeStruct((cnext, 2), _F32),
            ],
        )(h, s, Wf, Wd, Wn)

    h2, s3 = mid(h1, s2, W2f, W2d, W3f, 64, 64, 64)
    h3, s4 = mid(h2, s3, W3f, W3d, W4f, 64, 64, 128)
    h4, s5 = mid(h3, s4, W4f, W4d, W5, 64, 128, 1024)

    out = pl.pallas_call(
        functools.partial(_final_kernel, cnt=float(_B * _N)),
        grid=grid,
        in_specs=[hblk(128), full((1024, 2)), full((1024, 128))],
        out_specs=pl.BlockSpec((1, 1024, 3), lambda b, i: (b, 0, 0)),
        out_shape=jax.ShapeDtypeStruct((_B, 1024, 3), _F32),
    )(h4, s5, W5)
    return out


# speedup vs baseline: 6.6499x; 1.6498x over previous
import functools

import jax
import jax.numpy as jnp
from jax.experimental import pallas as pl

EPS = 1e-6
_B, _N, _K = 4, 2048, 20
_BNQ = 256
_BN = 512
_X = _B * _N
_F32 = jnp.float32


def _knn_feat_kernel(x_ref, xq_ref, xhi_ref, xmd_ref, xlo_ref, W1f_ref,
                     feat_ref, s1_ref):
    b = pl.program_id(0)
    i = pl.program_id(1)

    @pl.when(jnp.logical_and(b == 0, i == 0))
    def _init():
        s1_ref[...] = jnp.zeros(s1_ref.shape, _F32)

    xall = x_ref[0]
    ctr = xq_ref[0]

    inner = jnp.dot(xall, ctr, preferred_element_type=_F32)
    sq_all = jnp.sum(xall * xall, axis=1, keepdims=True)
    sq_q = jnp.sum(ctr * ctr, axis=0, keepdims=True)
    pdT = (2.0 * inner - sq_q) - sq_all

    iota = jax.lax.broadcasted_iota(jnp.int32, (_N, _BNQ), 0)
    c = [ctr[0:1], ctr[1:2], ctr[2:3]]
    xhi, xmd, xlo = xhi_ref[0], xmd_ref[0], xlo_ref[0]

    acc_s = jnp.zeros((64, 1), _F32)
    acc_q = jnp.zeros((64, 1), _F32)
    for j in range(_K):
        idx = jnp.argmax(pdT, axis=0)[None, :]
        eq = iota == idx
        pdT = jnp.where(eq, -1e30, pdT)
        ohb = eq.astype(jnp.bfloat16)
        nbrT = (jnp.dot(xhi, ohb, preferred_element_type=_F32)
                + jnp.dot(xmd, ohb, preferred_element_type=_F32)
                + jnp.dot(xlo, ohb, preferred_element_type=_F32))
        n = [nbrT[0:1], nbrT[1:2], nbrT[2:3]]
        r = [n[1] * c[2] - n[2] * c[1],
             n[2] * c[0] - n[0] * c[2],
             n[0] * c[1] - n[1] * c[0]]
        f = jnp.concatenate(
            [n[0] - c[0], c[0], r[0],
             n[1] - c[1], c[1], r[1],
             n[2] - c[2], c[2], r[2]], axis=0)
        feat_ref[0, j] = f
        ps = [jnp.dot(W1f_ref[...], f[3 * d:3 * d + 3],
                      preferred_element_type=_F32) for d in range(3)]
        nsq = ps[0] * ps[0] + ps[1] * ps[1] + ps[2] * ps[2]
        nrm = jnp.sqrt(nsq) + EPS
        acc_s += jnp.sum(nrm, axis=1, keepdims=True)
        acc_q += jnp.sum(nrm * nrm, axis=1, keepdims=True)
    s1_ref[...] += jnp.concatenate([acc_s, acc_q], axis=1)


def _lbr(ps, ds, mean, denom):
    nsq = ps[0] * ps[0] + ps[1] * ps[1] + ps[2] * ps[2]
    nrm = jnp.sqrt(nsq) + EPS
    bn = (nrm - mean) / denom
    ps = [p / nrm * bn for p in ps]
    dot = ps[0] * ds[0] + ps[1] * ds[1] + ps[2] * ds[2]
    dsq = ds[0] * ds[0] + ds[1] * ds[1] + ds[2] * ds[2]
    coef = jnp.where(dot >= 0, 0.0, dot / (dsq + EPS))
    return [ps[d] - coef * ds[d] for d in range(3)]


def _norm_stats(ps):
    nsq = ps[0] * ps[0] + ps[1] * ps[1] + ps[2] * ps[2]
    nrm = jnp.sqrt(nsq) + EPS
    return jnp.concatenate(
        [jnp.sum(nrm, axis=1, keepdims=True),
         jnp.sum(nrm * nrm, axis=1, keepdims=True)], axis=1)


def _bn_coefs(s_ref, cnt):
    mean = s_ref[:, 0:1] / cnt
    var = s_ref[:, 1:2] / cnt - mean * mean
    return mean, jnp.sqrt(var + 1e-5)


def _layer1_kernel(feat_ref, s1_ref, W1f_ref, W1d_ref, W2f_ref, h1_ref, s2_ref):
    b = pl.program_id(0)
    i = pl.program_id(1)

    @pl.when(jnp.logical_and(b == 0, i == 0))
    def _init():
        s2_ref[...] = jnp.zeros(s2_ref.shape, _F32)

    mean, denom = _bn_coefs(s1_ref, float(_B * _N * _K))
    bn = feat_ref.shape[3]
    acc = [jnp.zeros((64, bn), _F32) for _ in range(3)]
    for j in range(_K):
        f = feat_ref[0, j]
        ps = [jnp.dot(W1f_ref[...], f[3 * d:3 * d + 3],
                      preferred_element_type=_F32) for d in range(3)]
        ds = [jnp.dot(W1d_ref[...], f[3 * d:3 * d + 3],
                      preferred_element_type=_F32) for d in range(3)]
        out = _lbr(ps, ds, mean, denom)
        for d in range(3):
            acc[d] += out[d]
    hs = [a / float(_K) for a in acc]
    for d in range(3):
        h1_ref[0, d] = hs[d]
    p2 = [jnp.dot(W2f_ref[...], hs[d], preferred_element_type=_F32)
          for d in range(3)]
    s2_ref[...] += _norm_stats(p2)


def _mid_kernel(h_ref, s_ref, Wf_ref, Wd_ref, Wn_ref, ho_ref, sn_ref, *, cnt):
    b = pl.program_id(0)
    i = pl.program_id(1)

    @pl.when(jnp.logical_and(b == 0, i == 0))
    def _init():
        sn_ref[...] = jnp.zeros(sn_ref.shape, _F32)

    mean, denom = _bn_coefs(s_ref, cnt)
    hs = [h_ref[0, d] for d in range(3)]
    ps = [jnp.dot(Wf_ref[...], hs[d], preferred_element_type=_F32)
          for d in range(3)]
    ds = [jnp.dot(Wd_ref[...], hs[d], preferred_element_type=_F32)
          for d in range(3)]
    out = _lbr(ps, ds, mean, denom)
    for d in range(3):
        ho_ref[0, d] = out[d]
    pn = [jnp.dot(Wn_ref[...], out[d], preferred_element_type=_F32)
          for d in range(3)]
    sn_ref[...] += _norm_stats(pn)


def _final_kernel(h_ref, s5_ref, W5_ref, out_ref, *, cnt):
    i = pl.program_id(1)

    @pl.when(i == 0)
    def _init():
        out_ref[...] = jnp.zeros(out_ref.shape, _F32)

    mean, denom = _bn_coefs(s5_ref, cnt)
    hs = [h_ref[0, d] for d in range(3)]
    ps = [jnp.dot(W5_ref[...], hs[d], preferred_element_type=_F32)
          for d in range(3)]
    nsq = ps[0] * ps[0] + ps[1] * ps[1] + ps[2] * ps[2]
    nrm = jnp.sqrt(nsq) + EPS
    bn = (nrm - mean) / denom
    for d in range(3):
        col = jnp.sum(ps[d] / nrm * bn, axis=1, keepdims=True) / float(_N)
        out_ref[0, :, d:d + 1] += col


def kernel(x, W1f, W1d, W2f, W2d, W3f, W3d, W4f, W4d, W5):
    xT = jnp.transpose(x, (0, 2, 1))
    xhi = xT.astype(jnp.bfloat16)
    r1 = xT - xhi.astype(_F32)
    xmd = r1.astype(jnp.bfloat16)
    xlo = (r1 - xmd.astype(_F32)).astype(jnp.bfloat16)

    full = lambda shape: pl.BlockSpec(shape, lambda b, i: (0,) * len(shape))
    perb = lambda shape: pl.BlockSpec(
        shape, lambda b, i: (b,) + (0,) * (len(shape) - 1))

    feat, s1 = pl.pallas_call(
        _knn_feat_kernel,
        grid=(_B, _N // _BNQ),
        in_specs=[
            perb((1, _N, 3)),
            pl.BlockSpec((1, 3, _BNQ), lambda b, i: (b, 0, i)),
            perb((1, 3, _N)), perb((1, 3, _N)), perb((1, 3, _N)),
            full((64, 3)),
        ],
        out_specs=[
            pl.BlockSpec((1, _K, 9, _BNQ), lambda b, i: (b, 0, 0, i)),
            full((64, 2)),
        ],
        out_shape=[
            jax.ShapeDtypeStruct((_B, _K, 9, _N), _F32),
            jax.ShapeDtypeStruct((64, 2), _F32),
        ],
    )(x, xT, xhi, xmd, xlo, W1f)

    grid = (_B, _N // _BN)
    hblk = lambda c: pl.BlockSpec((1, 3, c, _BN), lambda b, i: (b, 0, 0, i))

    h1, s2 = pl.pallas_call(
        _layer1_kernel,
        grid=grid,
        in_specs=[
            pl.BlockSpec((1, _K, 9, _BN), lambda b, i: (b, 0, 0, i)),
            full((64, 2)), full((64, 3)), full((64, 3)), full((64, 64)),
        ],
        out_specs=[hblk(64), full((64, 2))],
        out_shape=[
            jax.ShapeDtypeStruct((_B, 3, 64, _N), _F32),
            jax.ShapeDtypeStruct((64, 2), _F32),
        ],
    )(feat, s1, W1f, W1d, W2f)

    def mid(h, s, Wf, Wd, Wn, cin, cout, cnext):
        return pl.pallas_call(
            functools.partial(_mid_kernel, cnt=float(_B * _N)),
            grid=grid,
            in_specs=[
                hblk(cin), full((cout, 2)),
                full((cout, cin)), full((cout, cin)), full((cnext, cout)),
            ],
            out_specs=[hblk(cout), full((cnext, 2))],
            out_shape=[
                jax.ShapeDtypeStruct((_B, 3, cout, _N), _F32),
                jax.ShapeDtypeStruct((cnext, 2), _F32),
            ],
        )(h, s, Wf, Wd, Wn)

    h2, s3 = mid(h1, s2, W2f, W2d, W3f, 64, 64, 64)
    h3, s4 = mid(h2, s3, W3f, W3d, W4f, 64, 64, 128)
    h4, s5 = mid(h3, s4, W4f, W4d, W5, 64, 128, 1024)

    out = pl.pallas_call(
        functools.partial(_final_kernel, cnt=float(_B * _N)),
        grid=grid,
        in_specs=[hblk(128), full((1024, 2)), full((1024, 128))],
        out_specs=pl.BlockSpec((1, 1024, 3), lambda b, i: (b, 0, 0)),
        out_shape=jax.ShapeDtypeStruct((_B, 1024, 3), _F32),
    )(h4, s5, W5)
    return out
